# baseline (speedup 1.0000x reference)
import functools

import jax
import jax.numpy as jnp
from jax import lax
from jax.experimental import pallas as pl
from jax.experimental.pallas import tpu as pltpu
from jax.experimental.pallas import tpu_sc as plsc

K = 10
KPAD = 8
RBLK = 256
PBLK = 512
BF = jnp.bfloat16


def _leaky(z):
    return jnp.where(z >= 0, z, 0.2 * z)


def _knn_body(x_rows_ref, x_full_ref, idx_ref, t_ref):
    xf = x_full_ref[0]
    xr = x_rows_ref[0]
    n = t_ref.shape[0]
    inner = lax.dot_general(xf.astype(BF), xr.astype(BF),
                            (((0,), (0,)), ((), ())),
                            preferred_element_type=jnp.float32)
    sq_j = jnp.sum(xf * xf, axis=0, keepdims=True)
    sq_cand = sq_j.reshape(n, 1)
    sq_query = jnp.sum(xr * xr, axis=0, keepdims=True)
    t = -((sq_query - 2.0 * inner) + sq_cand)
    r = t.shape[1]
    col = (lax.broadcasted_iota(jnp.int32, (1, r), 1)
           + pl.program_id(1) * r)
    iota0 = lax.broadcasted_iota(jnp.int32, t.shape, 0)
    t = jnp.where(iota0 == col, -jnp.inf, t)
    t_ref[...] = t
    m = jnp.max(t, axis=0, keepdims=True)
    ii = lax.broadcasted_iota(jnp.int32, (1, n), 1)
    hilo = jnp.concatenate(
        [(ii >> 6).astype(jnp.float32), (ii & 63).astype(jnp.float32)],
        axis=0).astype(BF)
    pos = [col[0]]
    for k in range(1, K):
        tcur = t_ref[...]
        eq = tcur >= m
        z = jnp.where(eq, 1.0, 0.0).astype(BF)
        hl = lax.dot_general(hilo, z, (((1,), (0,)), ((), ())),
                             preferred_element_type=jnp.float32)
        p = hl[0:1] * 64.0 + hl[1:2]
        pos.append(p.astype(jnp.int32)[0])
        if k < K - 1:
            t2 = jnp.where(eq, -jnp.inf, tcur)
            t_ref[...] = t2
            m = jnp.max(t2, axis=0, keepdims=True)
    idx_ref[0, 0, :] = pos[0] | (pos[8] << 16)
    idx_ref[0, 1, :] = pos[1] | (pos[9] << 16)
    for k in range(2, 8):
        idx_ref[0, k, :] = pos[k]


def _knn_topk(x):
    b, c, n = x.shape
    grid = (b, n // RBLK)
    return pl.pallas_call(
        _knn_body,
        grid=grid,
        in_specs=[
            pl.BlockSpec((1, c, RBLK), lambda i, j: (i, 0, j)),
            pl.BlockSpec((1, c, n), lambda i, j: (i, 0, 0)),
        ],
        out_specs=pl.BlockSpec((1, KPAD, RBLK), lambda i, j: (i, 0, j)),
        out_shape=jax.ShapeDtypeStruct((b, KPAD, n), jnp.int32),
        scratch_shapes=[pltpu.VMEM((n, RBLK), jnp.float32)],
    )(x, x)


def _gather_sc(x, idx):
    b, c, n = x.shape
    ngb = c // 8
    mesh = plsc.VectorSubcoreMesh(core_axis_name="c", subcore_axis_name="s")

    @functools.partial(
        pl.kernel,
        mesh=mesh,
        out_type=jax.ShapeDtypeStruct((b, K * c, n), jnp.float32),
        scratch_types=[
            pltpu.VMEM((8, n), jnp.float32),
            pltpu.VMEM((KPAD, n), jnp.int32),
            pltpu.VMEM((8, n), jnp.float32),
        ],
        compiler_params=pltpu.CompilerParams(needs_layout_passes=False),
    )
    def run(x_hbm, idx_hbm, nbr_hbm, x_slab, idx_slab, out_slab):
        wid = lax.axis_index("s") * 2 + lax.axis_index("c")
        bb = wid // ngb
        cg = wid % ngb
        pltpu.sync_copy(idx_hbm.at[bb], idx_slab)
        pltpu.sync_copy(x_hbm.at[bb, pl.ds(cg * 8, 8)], x_slab)
        for k in range(K):
            @plsc.parallel_loop(0, n // 16, unroll=4)
            def nb_body(nb):
                base = nb * 16
                if k < KPAD:
                    iv = idx_slab[k, pl.ds(base, 16)] & jnp.int32(0xFFFF)
                else:
                    iv = lax.shift_right_logical(
                        idx_slab[k - KPAD, pl.ds(base, 16)], 16)
                for cc in range(8):
                    g = plsc.load_gather(
                        x_slab, [jnp.full((16,), cc, jnp.int32), iv])
                    out_slab[cc, pl.ds(base, 16)] = g
            pltpu.sync_copy(
                out_slab, nbr_hbm.at[bb, pl.ds(k * c + cg * 8, 8)])

    return run(x, idx)


def _edge_body(wc_ref, wn_ref, b_ref, x_ref, nbr_ref, out_ref):
    xc = x_ref[0]
    c = xc.shape[0]
    zc = lax.dot_general(wc_ref[...], xc.astype(BF),
                         (((1,), (0,)), ((), ())),
                         preferred_element_type=jnp.float32)
    acc = None
    for k in range(K):
        nk = nbr_ref[0, k * c:(k + 1) * c, :]
        d = (nk - xc).astype(BF)
        zk = lax.dot_general(wn_ref[...], d, (((1,), (0,)), ((), ())),
                             preferred_element_type=jnp.float32)
        s = (zc + zk) + b_ref[...]
        acc = s if acc is None else jnp.maximum(acc, s)
    out_ref[0] = _leaky(acc)


def _edge(wc, wn, bias, x, nbr):
    b, c, n = x.shape
    o = wc.shape[0]
    grid = (b, n // PBLK)
    return pl.pallas_call(
        _edge_body,
        grid=grid,
        in_specs=[
            pl.BlockSpec((o, c), lambda i, j: (0, 0)),
            pl.BlockSpec((o, c), lambda i, j: (0, 0)),
            pl.BlockSpec((o, 1), lambda i, j: (0, 0)),
            pl.BlockSpec((1, c, PBLK), lambda i, j: (i, 0, j)),
            pl.BlockSpec((1, K * c, PBLK), lambda i, j: (i, 0, j)),
        ],
        out_specs=pl.BlockSpec((1, o, PBLK), lambda i, j: (i, 0, j)),
        out_shape=jax.ShapeDtypeStruct((b, o, n), jnp.float32),
    )(wc, wn, bias, x, nbr)


def _final_body(w_ref, b_ref, x0_ref, x1_ref, x2_ref, out_ref):
    cat = jnp.concatenate(
        [x0_ref[0], x1_ref[0], x2_ref[0]], axis=0).astype(BF)
    z = lax.dot_general(w_ref[...], cat, (((1,), (0,)), ((), ())),
                        preferred_element_type=jnp.float32)
    out_ref[0] = _leaky(z + b_ref[...])


def _final(w3, b3, x0, x1, x2):
    b, f, n = x0.shape
    o, ctot = w3.shape
    o2 = x2.shape[1]
    grid = (b, n // PBLK)
    return pl.pallas_call(
        _final_body,
        grid=grid,
        in_specs=[
            pl.BlockSpec((o, ctot), lambda i, j: (0, 0)),
            pl.BlockSpec((o, 1), lambda i, j: (0, 0)),
            pl.BlockSpec((1, f, PBLK), lambda i, j: (i, 0, j)),
            pl.BlockSpec((1, f, PBLK), lambda i, j: (i, 0, j)),
            pl.BlockSpec((1, o2, PBLK), lambda i, j: (i, 0, j)),
        ],
        out_specs=pl.BlockSpec((1, o, PBLK), lambda i, j: (i, 0, j)),
        out_shape=jax.ShapeDtypeStruct((b, o, n), jnp.float32),
    )(w3, b3, x0, x1, x2)


def kernel(features, W1, b1, W2, b2, W3, b3):
    x0 = features
    f = W1.shape[0]
    w1c = W1[:, :f].astype(BF)
    w1n = W1[:, f:].astype(BF)
    w2c = W2[:, :f].astype(BF)
    w2n = W2[:, f:].astype(BF)
    w3 = W3.astype(BF)
    b1c, b2c, b3c = b1[:, None], b2[:, None], b3[:, None]

    idx1 = _knn_topk(x0)
    nbr1 = _gather_sc(x0, idx1)
    x1 = _edge(w1c, w1n, b1c, x0, nbr1)
    idx2 = _knn_topk(x1)
    nbr2 = _gather_sc(x1, idx2)
    x2 = _edge(w2c, w2n, b2c, x1, nbr2)
    return _final(w3, b3c, x0, x1, x2)


# speedup vs baseline: 1.5070x; 1.5070x over previous
import functools

import jax
import jax.numpy as jnp
from jax import lax
from jax.experimental import pallas as pl
from jax.experimental.pallas import tpu as pltpu
from jax.experimental.pallas import tpu_sc as plsc

K = 10
KPAD = 8
RBLK = 512
PBLK = 512
BF = jnp.bfloat16


def _leaky(z):
    return jnp.where(z >= 0, z, 0.2 * z)


def _knn_body(x_rows_ref, x_full_ref, idx_ref, t_ref):
    xf = x_full_ref[0]
    xr = x_rows_ref[0]
    n = t_ref.shape[0]
    inner = lax.dot_general(xf.astype(BF), xr.astype(BF),
                            (((0,), (0,)), ((), ())),
                            preferred_element_type=jnp.float32)
    sq_j = jnp.sum(xf * xf, axis=0, keepdims=True)
    sq_cand = sq_j.reshape(n, 1)
    sq_query = jnp.sum(xr * xr, axis=0, keepdims=True)
    t = -((sq_query - 2.0 * inner) + sq_cand)
    r = t.shape[1]
    col = (lax.broadcasted_iota(jnp.int32, (1, r), 1)
           + pl.program_id(1) * r)
    iota0 = lax.broadcasted_iota(jnp.int32, t.shape, 0)
    t = jnp.where(iota0 == col, -jnp.inf, t)
    t_ref[...] = t
    m = jnp.max(t, axis=0, keepdims=True)
    ii = lax.broadcasted_iota(jnp.int32, (1, n), 1)
    hilo = jnp.concatenate(
        [(ii >> 6).astype(jnp.float32), (ii & 63).astype(jnp.float32)],
        axis=0).astype(BF)
    pos = [col[0]]
    for k in range(1, K):
        tcur = t_ref[...]
        eq = tcur >= m
        z = jnp.where(eq, 1.0, 0.0).astype(BF)
        hl = jnp.concatenate(
            [lax.dot_general(hilo, z[:, :256], (((1,), (0,)), ((), ())),
                             preferred_element_type=jnp.float32),
             lax.dot_general(hilo, z[:, 256:], (((1,), (0,)), ((), ())),
                             preferred_element_type=jnp.float32)],
            axis=1)
        p = hl[0:1] * 64.0 + hl[1:2]
        pos.append(p.astype(jnp.int32)[0])
        if k < K - 1:
            t2 = jnp.where(eq, -jnp.inf, tcur)
            t_ref[...] = t2
            m = jnp.max(t2, axis=0, keepdims=True)
    idx_ref[0, 0, :] = pos[0] | (pos[8] << 16)
    idx_ref[0, 1, :] = pos[1] | (pos[9] << 16)
    for k in range(2, 8):
        idx_ref[0, k, :] = pos[k]


def _knn_topk(x):
    b, c, n = x.shape
    grid = (b, n // RBLK)
    return pl.pallas_call(
        _knn_body,
        grid=grid,
        in_specs=[
            pl.BlockSpec((1, c, RBLK), lambda i, j: (i, 0, j)),
            pl.BlockSpec((1, c, n), lambda i, j: (i, 0, 0)),
        ],
        out_specs=pl.BlockSpec((1, KPAD, RBLK), lambda i, j: (i, 0, j)),
        out_shape=jax.ShapeDtypeStruct((b, KPAD, n), jnp.int32),
        scratch_shapes=[pltpu.VMEM((n, RBLK), jnp.float32)],
    )(x, x)


def _gather_sc(x, idx):
    b, c, n = x.shape
    assert b == 1
    ngb = c // 8
    nq = 32 // ngb
    qn = n // nq
    mesh = plsc.VectorSubcoreMesh(core_axis_name="c", subcore_axis_name="s")

    @functools.partial(
        pl.kernel,
        mesh=mesh,
        out_type=jax.ShapeDtypeStruct((1, K * c, n), jnp.float32),
        scratch_types=[
            pltpu.VMEM((8, n), jnp.float32),
            pltpu.VMEM((KPAD, qn), jnp.int32),
            pltpu.VMEM((8, qn), jnp.float32),
        ],
        compiler_params=pltpu.CompilerParams(needs_layout_passes=False),
    )
    def run(x_hbm, idx_hbm, nbr_hbm, x_slab, idx_slab, out_slab):
        wid = lax.axis_index("s") * 2 + lax.axis_index("c")
        cg = wid % ngb
        q = wid // ngb
        pltpu.sync_copy(idx_hbm.at[0, :, pl.ds(q * qn, qn)], idx_slab)
        pltpu.sync_copy(x_hbm.at[0, pl.ds(cg * 8, 8)], x_slab)
        for k in range(K):
            @plsc.parallel_loop(0, qn // 16, unroll=4)
            def nb_body(nb):
                base = nb * 16
                if k < KPAD:
                    iv = idx_slab[k, pl.ds(base, 16)] & jnp.int32(0xFFFF)
                else:
                    iv = lax.shift_right_logical(
                        idx_slab[k - KPAD, pl.ds(base, 16)], 16)
                for cc in range(8):
                    g = plsc.load_gather(
                        x_slab, [jnp.full((16,), cc, jnp.int32), iv])
                    out_slab[cc, pl.ds(base, 16)] = g
            pltpu.sync_copy(
                out_slab,
                nbr_hbm.at[0, pl.ds(k * c + cg * 8, 8), pl.ds(q * qn, qn)])

    return run(x, idx)


def _edge_body(wc_ref, wn_ref, b_ref, x_ref, nbr_ref, out_ref):
    xc = x_ref[0]
    c = xc.shape[0]
    zc = lax.dot_general(wc_ref[...], xc.astype(BF),
                         (((1,), (0,)), ((), ())),
                         preferred_element_type=jnp.float32)
    acc = None
    for k in range(K):
        nk = nbr_ref[0, k * c:(k + 1) * c, :]
        d = (nk - xc).astype(BF)
        zk = lax.dot_general(wn_ref[...], d, (((1,), (0,)), ((), ())),
                             preferred_element_type=jnp.float32)
        s = (zc + zk) + b_ref[...]
        acc = s if acc is None else jnp.maximum(acc, s)
    out_ref[0] = _leaky(acc)


def _edge(wc, wn, bias, x, nbr):
    b, c, n = x.shape
    o = wc.shape[0]
    grid = (b, n // PBLK)
    return pl.pallas_call(
        _edge_body,
        grid=grid,
        in_specs=[
            pl.BlockSpec((o, c), lambda i, j: (0, 0)),
            pl.BlockSpec((o, c), lambda i, j: (0, 0)),
            pl.BlockSpec((o, 1), lambda i, j: (0, 0)),
            pl.BlockSpec((1, c, PBLK), lambda i, j: (i, 0, j)),
            pl.BlockSpec((1, K * c, PBLK), lambda i, j: (i, 0, j)),
        ],
        out_specs=pl.BlockSpec((1, o, PBLK), lambda i, j: (i, 0, j)),
        out_shape=jax.ShapeDtypeStruct((b, o, n), jnp.float32),
    )(wc, wn, bias, x, nbr)


def _final_body(w_ref, b_ref, x0_ref, x1_ref, x2_ref, out_ref):
    cat = jnp.concatenate(
        [x0_ref[0], x1_ref[0], x2_ref[0]], axis=0).astype(BF)
    z = lax.dot_general(w_ref[...], cat, (((1,), (0,)), ((), ())),
                        preferred_element_type=jnp.float32)
    out_ref[0] = _leaky(z + b_ref[...])


def _final(w3, b3, x0, x1, x2):
    b, f, n = x0.shape
    o, ctot = w3.shape
    o2 = x2.shape[1]
    grid = (b, n // PBLK)
    return pl.pallas_call(
        _final_body,
        grid=grid,
        in_specs=[
            pl.BlockSpec((o, ctot), lambda i, j: (0, 0)),
            pl.BlockSpec((o, 1), lambda i, j: (0, 0)),
            pl.BlockSpec((1, f, PBLK), lambda i, j: (i, 0, j)),
            pl.BlockSpec((1, f, PBLK), lambda i, j: (i, 0, j)),
            pl.BlockSpec((1, o2, PBLK), lambda i, j: (i, 0, j)),
        ],
        out_specs=pl.BlockSpec((1, o, PBLK), lambda i, j: (i, 0, j)),
        out_shape=jax.ShapeDtypeStruct((b, o, n), jnp.float32),
    )(w3, b3, x0, x1, x2)


def kernel(features, W1, b1, W2, b2, W3, b3):
    x0 = features
    f = W1.shape[0]
    w1c = W1[:, :f].astype(BF)
    w1n = W1[:, f:].astype(BF)
    w2c = W2[:, :f].astype(BF)
    w2n = W2[:, f:].astype(BF)
    w3 = W3.astype(BF)
    b1c, b2c, b3c = b1[:, None], b2[:, None], b3[:, None]

    outs = []
    for b in range(x0.shape[0]):
        xb = x0[b:b + 1]
        idx1 = _knn_topk(xb)
        nbr1 = _gather_sc(xb, idx1)
        x1 = _edge(w1c, w1n, b1c, xb, nbr1)
        idx2 = _knn_topk(x1)
        nbr2 = _gather_sc(x1, idx2)
        x2 = _edge(w2c, w2n, b2c, x1, nbr2)
        outs.append(_final(w3, b3c, xb, x1, x2))
    return jnp.concatenate(outs, axis=0)
